# baseline (speedup 1.0000x reference)
import functools

import jax
import jax.numpy as jnp
from jax import lax
from jax.experimental import pallas as pl
from jax.experimental.pallas import tpu as pltpu
from jax.experimental.pallas import tpu_sc as plsc

NUM_CHAR_TYPES = 1000
E = 32
D = 128
LOC = 4
NMASK = 8
N = 1024 * 4 * 50

NC, NS, L = 2, 16, 16
NW = NC * NS
RPW = N // NW
C = 64
NCHUNK = RPW // C
GCH = 20
NG = NCHUNK // GCH
GROUP_I = GCH * C * LOC
GROUP_M = GCH * C * NMASK


def _project_body(emb_ref, w_ref, b_ref, out_ref):
    part = jnp.dot(emb_ref[...], w_ref[...], preferred_element_type=jnp.float32)
    sel = jnp.where(pl.program_id(0) == 0, 1.0, 0.0)
    out_ref[...] = part + sel * b_ref[...]


def _project_tables(emb_table, We, b2):
    return pl.pallas_call(
        _project_body,
        grid=(LOC,),
        in_specs=[
            pl.BlockSpec((NUM_CHAR_TYPES, E), lambda l: (0, 0)),
            pl.BlockSpec((E, D), lambda l: (l, 0)),
            pl.BlockSpec((1, D), lambda l: (0, 0)),
        ],
        out_specs=pl.BlockSpec((NUM_CHAR_TYPES, D), lambda l: (l, 0)),
        out_shape=jax.ShapeDtypeStruct((LOC * NUM_CHAR_TYPES, D),
                                       jnp.float32),
    )(emb_table, We, b2)


def _sc_body(p_hbm, chars_hbm, masks_hbm, wm_hbm, out_hbm,
             idxg, maskg, gbuf, wm_v, out_v, sg0, sg1, ss0, ss1):
    wid = lax.axis_index("s") * NC + lax.axis_index("c")
    wbase = wid * RPW
    sem_g = (sg0, sg1)
    sem_s = (ss0, ss1)

    pltpu.sync_copy(wm_hbm, wm_v)
    offpat = (lax.iota(jnp.int32, L) % LOC) * NUM_CHAR_TYPES

    def load_group(g):
        gb = g % 2
        pltpu.sync_copy(chars_hbm.at[pl.ds(wbase * LOC + g * GROUP_I, GROUP_I)],
                        idxg.at[gb])
        pltpu.sync_copy(masks_hbm.at[pl.ds(wbase * NMASK + g * GROUP_M,
                                           GROUP_M)], maskg.at[gb])

        @plsc.parallel_loop(0, GROUP_I // L, 1, unroll=8)
        def _(i):
            sl = pl.ds(i * L, L)
            idxg[gb, sl] = idxg[gb, sl] + offpat

    def fire_gathers(ci, b):
        gb = (ci // GCH) % 2
        off = (ci % GCH) * (C * LOC)
        for h in range(2):
            pltpu.async_copy(
                p_hbm.at[idxg.at[gb, pl.ds(off + h * 128, 128)]],
                gbuf.at[b, pl.ds(h * 128, 128), :], sem_g[b])

    def wait_gathers(ci, b):
        gb = (ci // GCH) % 2
        off = (ci % GCH) * (C * LOC)
        for h in range(2):
            pltpu.make_async_copy(
                p_hbm.at[idxg.at[gb, pl.ds(off + h * 128, 128)]],
                gbuf.at[b, pl.ds(h * 128, 128), :], sem_g[b]).wait()

    def wait_store(b):
        pltpu.make_async_copy(out_v.at[b], out_hbm.at[0], sem_s[b]).wait()

    def compute(ci, b):
        gb = (ci // GCH) % 2
        moff = (ci % GCH) * (C * NMASK)
        for half in range(2):
            wv = [[wm_v[j, pl.ds(half * 64 + k * L, L)] for k in range(4)]
                  for j in range(NMASK)]

            @plsc.parallel_loop(0, C // 2, 1, unroll=4)
            def row_body(r2, wv=wv, half=half):
                mv = maskg[gb, pl.ds(moff + r2 * (2 * NMASK), 2 * NMASK)]
                for rr in range(2):
                    r = r2 * 2 + rr
                    r4 = r * LOC
                    msp = [jnp.full((L,), mv[rr * NMASK + j], jnp.float32)
                           for j in range(NMASK)]
                    for k in range(4):
                        sl = pl.ds(half * 64 + k * L, L)
                        g01 = gbuf[b, r4, sl] + gbuf[b, r4 + 1, sl]
                        g23 = gbuf[b, r4 + 2, sl] + gbuf[b, r4 + 3, sl]
                        p = [msp[j] * wv[j][k] for j in range(NMASK)]
                        s0 = (p[0] + p[1]) + (p[2] + p[3])
                        s1 = (p[4] + p[5]) + (p[6] + p[7])
                        out_v[b, r, sl] = (g01 + g23) + (s0 + s1)

    load_group(0)
    fire_gathers(0, 0)

    def pair_body(cp, _):
        for b in range(2):
            ci = cp * 2 + b
            nci = ci + 1

            @pl.when(jnp.logical_and(nci < NCHUNK, nci % GCH == 0))
            def _():
                load_group(nci // GCH)

            @pl.when(nci < NCHUNK)
            def _():
                fire_gathers(nci, 1 - b)

            wait_gathers(ci, b)

            @pl.when(ci >= 2)
            def _():
                wait_store(b)

            compute(ci, b)
            pltpu.async_copy(out_v.at[b], out_hbm.at[wid * NCHUNK + ci],
                             sem_s[b])
        return 0

    lax.fori_loop(0, NCHUNK // 2, pair_body, 0)
    wait_store(0)
    wait_store(1)


@jax.jit
def _sc_encode(P, chars2d, masksf, Wm):
    mesh = plsc.VectorSubcoreMesh(core_axis_name="c", subcore_axis_name="s",
                                  num_cores=NC, num_subcores=NS)
    run = functools.partial(
        pl.kernel,
        out_type=jax.ShapeDtypeStruct((N // C, C, D), jnp.float32),
        mesh=mesh,
        compiler_params=pltpu.CompilerParams(needs_layout_passes=False),
        scratch_types=[
            pltpu.VMEM((2, GROUP_I), jnp.int32),
            pltpu.VMEM((2, GROUP_M), jnp.float32),
            pltpu.VMEM((2, LOC * C, D), jnp.float32),
            pltpu.VMEM((NMASK, D), jnp.float32),
            pltpu.VMEM((2, C, D), jnp.float32),
            pltpu.SemaphoreType.DMA,
            pltpu.SemaphoreType.DMA,
            pltpu.SemaphoreType.DMA,
            pltpu.SemaphoreType.DMA,
        ],
    )(_sc_body)
    return run(P, chars2d, masksf, Wm)


def kernel(chars, masks, emb_table, W, b):
    B, EX, S, _ = chars.shape
    P = _project_tables(emb_table, W[: LOC * E], b.reshape(1, D))
    chars2d = chars.reshape(N * LOC).astype(jnp.int32)
    masksf = masks.reshape(N * NMASK)
    Wm = W[LOC * E:]
    out = _sc_encode(P, chars2d, masksf, Wm)
    return out.reshape(B, EX, S, D)


# speedup vs baseline: 1.2055x; 1.2055x over previous
import functools

import jax
import jax.numpy as jnp
from jax import lax
from jax.experimental import pallas as pl
from jax.experimental.pallas import tpu as pltpu
from jax.experimental.pallas import tpu_sc as plsc

NUM_CHAR_TYPES = 1000
E = 32
D = 128
LOC = 4
NMASK = 8
N = 1024 * 4 * 50

NC, NS, L = 2, 16, 16
NW = NC * NS
RPW = N // NW
C = 128
NCHUNK = RPW // C
GCH = 10
NG = NCHUNK // GCH
GROUP_C = GCH * C
GROUP_M = GROUP_C * NMASK


def _project_body(emb_ref, w_ref, b_ref, out_ref):
    part = jnp.dot(emb_ref[...], w_ref[...], preferred_element_type=jnp.float32)
    sel = jnp.where(pl.program_id(0) == 0, 1.0, 0.0)
    out_ref[...] = part + sel * b_ref[...]


def _project_tables(emb_table, We, b2):
    return pl.pallas_call(
        _project_body,
        grid=(LOC,),
        in_specs=[
            pl.BlockSpec((NUM_CHAR_TYPES, E), lambda l: (0, 0)),
            pl.BlockSpec((E, D), lambda l: (l, 0)),
            pl.BlockSpec((1, D), lambda l: (0, 0)),
        ],
        out_specs=pl.BlockSpec((NUM_CHAR_TYPES, D), lambda l: (l, 0)),
        out_shape=jax.ShapeDtypeStruct((LOC * NUM_CHAR_TYPES, D),
                                       jnp.float32),
    )(emb_table, We, b2)


def _sc_body(p_hbm, charst_hbm, masks_hbm, wm_hbm, out_hbm,
             idxg, maskg, gbuf, wm_v, sg0, sg1, ss0, ss1):
    wid = lax.axis_index("s") * NC + lax.axis_index("c")
    wbase = wid * RPW
    sem_g = (sg0, sg1)
    sem_s = (ss0, ss1)

    pltpu.sync_copy(wm_hbm, wm_v)

    def load_group(g):
        gb = g % 2
        for l in range(LOC):
            pltpu.sync_copy(
                charst_hbm.at[pl.ds(l * N + wbase + g * GROUP_C, GROUP_C)],
                idxg.at[gb, pl.ds(l * GROUP_C, GROUP_C)])
        pltpu.sync_copy(masks_hbm.at[pl.ds(wbase * NMASK + g * GROUP_M,
                                           GROUP_M)], maskg.at[gb])
        for l in range(LOC):
            if l == 0:
                continue
            off = jnp.full((L,), l * NUM_CHAR_TYPES, jnp.int32)

            @plsc.parallel_loop(0, GROUP_C // L, 1, unroll=8)
            def _(i, l=l, off=off):
                sl = pl.ds(l * GROUP_C + i * L, L)
                idxg[gb, sl] = idxg[gb, sl] + off

    def fire_gather_adds(ci, b):
        gb = (ci // GCH) % 2
        coff = (ci % GCH) * C
        for l in range(LOC):
            pltpu.async_copy(
                p_hbm.at[idxg.at[gb, pl.ds(l * GROUP_C + coff, C)]],
                gbuf.at[b], sem_g[b], add=True)

    def wait_gather_adds(ci, b):
        gb = (ci // GCH) % 2
        coff = (ci % GCH) * C
        for l in range(LOC):
            pltpu.make_async_copy(
                p_hbm.at[idxg.at[gb, pl.ds(l * GROUP_C + coff, C)]],
                gbuf.at[b], sem_g[b]).wait()

    def wait_store(b):
        pltpu.make_async_copy(gbuf.at[b], out_hbm.at[0], sem_s[b]).wait()

    def mask_term(ci, b):
        gb = (ci // GCH) % 2
        moff = (ci % GCH) * (C * NMASK)
        for half in range(2):
            wv = [[wm_v[j, pl.ds(half * 64 + k * L, L)] for k in range(4)]
                  for j in range(NMASK)]

            @plsc.parallel_loop(0, C // 2, 1, unroll=2)
            def row_body(r2, wv=wv, half=half):
                mv = maskg[gb, pl.ds(moff + r2 * (2 * NMASK), 2 * NMASK)]
                for rr in range(2):
                    r = r2 * 2 + rr
                    msp = [jnp.full((L,), mv[rr * NMASK + j], jnp.float32)
                           for j in range(NMASK)]
                    for k in range(4):
                        sl = pl.ds(half * 64 + k * L, L)
                        t = [msp[j] * wv[j][k] for j in range(NMASK)]
                        s0 = (t[0] + t[1]) + (t[2] + t[3])
                        s1 = (t[4] + t[5]) + (t[6] + t[7])
                        gbuf[b, r, sl] = s0 + s1

    load_group(0)

    def pair_body(cp, _):
        for b in range(2):
            ci = cp * 2 + b

            @pl.when(jnp.logical_and(ci % GCH == 0, ci > 0))
            def _():
                load_group(ci // GCH)

            @pl.when(ci >= 2)
            def _():
                wait_store(b)

            mask_term(ci, b)
            fire_gather_adds(ci, b)

            @pl.when(ci >= 1)
            def _():
                wait_gather_adds(ci - 1, 1 - b)
                pltpu.async_copy(gbuf.at[1 - b],
                                 out_hbm.at[wid * NCHUNK + ci - 1],
                                 sem_s[1 - b])
        return 0

    lax.fori_loop(0, NCHUNK // 2, pair_body, 0)
    wait_gather_adds(NCHUNK - 1, 1)
    pltpu.async_copy(gbuf.at[1], out_hbm.at[wid * NCHUNK + NCHUNK - 1],
                     sem_s[1])
    wait_store(0)
    wait_store(1)


@jax.jit
def _sc_encode(P, charst, masksf, Wm):
    mesh = plsc.VectorSubcoreMesh(core_axis_name="c", subcore_axis_name="s",
                                  num_cores=NC, num_subcores=NS)
    run = functools.partial(
        pl.kernel,
        out_type=jax.ShapeDtypeStruct((N // C, C, D), jnp.float32),
        mesh=mesh,
        compiler_params=pltpu.CompilerParams(needs_layout_passes=False),
        scratch_types=[
            pltpu.VMEM((2, LOC * GROUP_C), jnp.int32),
            pltpu.VMEM((2, GROUP_M), jnp.float32),
            pltpu.VMEM((2, C, D), jnp.float32),
            pltpu.VMEM((NMASK, D), jnp.float32),
            pltpu.SemaphoreType.DMA,
            pltpu.SemaphoreType.DMA,
            pltpu.SemaphoreType.DMA,
            pltpu.SemaphoreType.DMA,
        ],
    )(_sc_body)
    return run(P, charst, masksf, Wm)


def kernel(chars, masks, emb_table, W, b):
    B, EX, S, _ = chars.shape
    P = _project_tables(emb_table, W[: LOC * E], b.reshape(1, D))
    charst = chars.reshape(N, LOC).T.reshape(N * LOC).astype(jnp.int32)
    masksf = masks.reshape(N * NMASK)
    Wm = W[LOC * E:]
    out = _sc_encode(P, charst, masksf, Wm)
    return out.reshape(B, EX, S, D)
